# baseline (speedup 1.0000x reference)
import jax
import jax.numpy as jnp
from jax import lax
from jax.experimental import pallas as pl
from jax.experimental.pallas import tpu as pltpu
from jax.experimental.pallas import tpu_sc as plsc

_N = 10000
_D = 128
_E = 320000
_NC = 2
_NS = 16
_NW = _NC * _NS
_K = 128
_CPP = 160
_C0 = 104
_C1 = _CPP - _C0
_EPAD = _NS * _CPP * _K
_EPT = _EPAD // _NS
_NPAD = 10240
_RPT = _NPAD // _NS
_WB_CHUNKS = ((0, 128), (128, 128), (256, 128), (384, 128), (512, 128))


def _deg_body(dst_hbm, ew_hbm, dis_hbm, dst_v, ew_v, part_v, shared, acc_v, tmp_v):
    cid = lax.axis_index("c")
    sid = lax.axis_index("s")

    @pl.when(cid == 0)
    def _():
        pltpu.sync_copy(dst_hbm.at[sid, 0], dst_v)
        pltpu.sync_copy(ew_hbm.at[sid, 0], ew_v)
        zeros = jnp.zeros((16,), jnp.float32)

        def zero_body(i, _):
            part_v[pl.ds(i * 16, 16)] = zeros
            return 0

        lax.fori_loop(0, _NPAD // 16, zero_body, 0)

        def scat_body(g, _):
            idx = dst_v[pl.ds(g * 16, 16)]
            w = ew_v[pl.ds(g * 16, 16)]
            plsc.addupdate_scatter(part_v, [idx], w)
            return 0

        lax.fori_loop(0, _EPT // 16, scat_body, 0)

        pltpu.sync_copy(part_v, shared.at[sid])
        plsc.subcore_barrier()

        base = sid * _RPT
        pltpu.sync_copy(shared.at[0, pl.ds(base, _RPT)], acc_v)
        for k in range(1, _NS):
            pltpu.sync_copy(shared.at[k, pl.ds(base, _RPT)], tmp_v)

            def add_body(i, _):
                sl = pl.ds(i * 16, 16)
                acc_v[sl] = acc_v[sl] + tmp_v[sl]
                return 0

            lax.fori_loop(0, _RPT // 16, add_body, 0)

        def newton_body(i, _):
            sl = pl.ds(i * 16, 16)
            xv = acc_v[sl] + 1.0
            bits = plsc.bitcast(xv, jnp.int32)
            y = plsc.bitcast(
                jnp.int32(0x5F3759DF) - lax.shift_right_logical(bits, 1),
                jnp.float32,
            )
            half = xv * 0.5
            for _u in range(3):
                y = y * (1.5 - half * y * y)
            acc_v[sl] = y
            return 0

        lax.fori_loop(0, _RPT // 16, newton_body, 0)
        pltpu.sync_copy(acc_v, dis_hbm.at[pl.ds(base, _RPT)])


_deg_kernel = pl.kernel(
    _deg_body,
    out_type=jax.ShapeDtypeStruct((_NPAD,), jnp.float32),
    mesh=plsc.VectorSubcoreMesh(core_axis_name="c", subcore_axis_name="s"),
    scratch_types=[
        pltpu.VMEM((_EPT,), jnp.int32),
        pltpu.VMEM((_EPT,), jnp.float32),
        pltpu.VMEM((_NPAD,), jnp.float32),
        pltpu.VMEM_SHARED((_NS, _NPAD), jnp.float32),
        pltpu.VMEM((_RPT,), jnp.float32),
        pltpu.VMEM((_RPT,), jnp.float32),
    ],
    compiler_params=pltpu.CompilerParams(needs_layout_passes=False),
)


def _bf16_splat(v):
    c16 = jnp.full((16,), 16, jnp.uint32)
    one = jnp.full((16,), 1, jnp.uint32)
    rnd = jnp.full((16,), 0x7FFF, jnp.uint32)
    vu = plsc.bitcast(v, jnp.uint32)
    vr = lax.shift_right_logical(
        vu + rnd + (lax.shift_right_logical(vu, c16) & one), c16)
    word = vr | lax.shift_left(vr, c16)
    return plsc.bitcast(word, jnp.bfloat16)


def _msg_body(g_hbm, src_hbm, dst_hbm, ew_hbm, out_hbm,
              src_v, dst_v, ew_v, buf, acc, gs0, gs1, ss0, ss1):
    cid = lax.axis_index("c")
    sid = lax.axis_index("s")
    gsem = (gs0, gs1)
    ssem = (ss0, ss1)

    zeros = jnp.zeros((32,), jnp.bfloat16)

    def zb(r, _):
        for cc in range(_D // 32):
            buf[0, r, pl.ds(cc * 32, 32)] = zeros
        return 0

    lax.fori_loop(0, _K, zb, 0)

    zbase = sid * _RPT
    for off, n in _WB_CHUNKS:
        pltpu.sync_copy(buf.at[0, pl.ds(0, n)], acc.at[pl.ds(zbase + off, n)])
    plsc.subcore_barrier()

    def _run(base, nc):
        pltpu.sync_copy(src_hbm.at[sid, pl.ds(base, nc)],
                        src_v.at[pl.ds(0, nc)])
        pltpu.sync_copy(dst_hbm.at[sid, pl.ds(base, nc)],
                        dst_v.at[pl.ds(0, nc)])
        pltpu.sync_copy(ew_hbm.at[sid, pl.ds(base, nc)],
                        ew_v.at[pl.ds(0, nc)])

        def _gather(c, b, sem):
            return pltpu.async_copy(g_hbm.at[src_v.at[c]], buf.at[b], sem)

        _gather(0, 0, gsem[0])
        _gather(1, 1, gsem[1])

        def chunk_body(i, _):
            for b in range(2):
                c = 2 * i + b
                pltpu.make_async_copy(g_hbm.at[src_v.at[c]],
                                      buf.at[b], gsem[b]).wait()
                cvec = jnp.full((16,), c, jnp.int32)

                def row_body(r, _):
                    ewf = plsc.load_gather(
                        ew_v, [cvec, jnp.full((16,), r, jnp.int32)])
                    ewb = _bf16_splat(ewf)
                    for k in range(_D // 32):
                        sl = pl.ds(32 * k, 32)
                        buf[b, r, sl] = buf[b, r, sl] * ewb
                    return 0

                lax.fori_loop(0, _K, row_body, 0)
                pltpu.async_copy(buf.at[b], acc.at[dst_v.at[c]],
                                 ssem[b], add=True)

                @pl.when(i < nc // 2 - 1)
                def _():
                    pltpu.make_async_copy(buf.at[b], acc.at[dst_v.at[c]],
                                          ssem[b]).wait()
                    _gather(c + 2, b, gsem[b])

            return 0

        lax.fori_loop(0, nc // 2, chunk_body, 0)
        for b in range(2):
            c = nc - 2 + b
            pltpu.make_async_copy(buf.at[b], acc.at[dst_v.at[c]],
                                  ssem[b]).wait()

    @pl.when(cid == 0)
    def _():
        _run(0, _C0)

    @pl.when(cid == 1)
    def _():
        _run(_C0, _C1)

    plsc.subcore_barrier()

    for off, n in _WB_CHUNKS:
        pltpu.sync_copy(acc.at[pl.ds(zbase + off, n)],
                        out_hbm.at[cid, pl.ds(zbase + off, n)])


_msg_kernel = pl.kernel(
    _msg_body,
    out_type=jax.ShapeDtypeStruct((_NC, _NPAD, _D), jnp.bfloat16),
    mesh=plsc.VectorSubcoreMesh(core_axis_name="c", subcore_axis_name="s"),
    scratch_types=[
        pltpu.VMEM((_C0, _K), jnp.int32),
        pltpu.VMEM((_C0, _K), jnp.int32),
        pltpu.VMEM((_C0, _K), jnp.float32),
        pltpu.VMEM((2, _K, _D), jnp.bfloat16),
        pltpu.VMEM_SHARED((_NPAD, _D), jnp.bfloat16),
        pltpu.SemaphoreType.DMA,
        pltpu.SemaphoreType.DMA,
        pltpu.SemaphoreType.DMA,
        pltpu.SemaphoreType.DMA,
    ],
    compiler_params=pltpu.CompilerParams(needs_layout_passes=False,
                                         use_tc_tiling_on_sc=False),
)


_GB = 2000
_G = _N // _GB


def _tc1_body(x_ref, w_ref, dis_ref, g_ref, gb_ref):
    h = jnp.dot(x_ref[...], w_ref[...], preferred_element_type=jnp.float32)
    g = h * dis_ref[...]
    g_ref[...] = g
    gb_ref[...] = g.astype(jnp.bfloat16)


def _tc2_body(p_ref, g_ref, dis_ref, b_ref, w_ref, g2_ref, g2b_ref):
    p = (p_ref[0].astype(jnp.float32) + p_ref[1].astype(jnp.float32))
    dis = dis_ref[...]
    s = (p + g_ref[...]) * dis + b_ref[...]
    t = jnp.maximum(s, 0.0)
    h = jnp.dot(t, w_ref[...], preferred_element_type=jnp.float32)
    g2 = h * dis
    g2_ref[...] = g2
    g2b_ref[...] = g2.astype(jnp.bfloat16)


def _tc3_body(p_ref, g_ref, dis_ref, b_ref, out_ref):
    p = (p_ref[0].astype(jnp.float32) + p_ref[1].astype(jnp.float32))
    out_ref[...] = (p + g_ref[...]) * dis_ref[...] + b_ref[...]


_bs_rows_f = pl.BlockSpec((_GB, _D), lambda i: (i, 0))
_bs_rows_b = pl.BlockSpec((_GB, _D), lambda i: (i, 0))
_bs_dis = pl.BlockSpec((_GB, 1), lambda i: (i, 0))
_bs_w = pl.BlockSpec((_D, _D), lambda i: (0, 0))
_bs_bias = pl.BlockSpec((1, _D), lambda i: (0, 0))
_bs_p = pl.BlockSpec((_NC, _GB, _D), lambda i: (0, i, 0))


def kernel(x, edge_index, edge_weight, W1, b1, W2, b2):
    src = edge_index[0].astype(jnp.int32)
    dst = edge_index[1].astype(jnp.int32)
    ew = edge_weight.astype(jnp.float32)

    pad = _EPAD - _E
    src_p = jnp.concatenate([src, jnp.zeros((pad,), jnp.int32)])
    dst_p = jnp.concatenate([dst, jnp.zeros((pad,), jnp.int32)])
    ew_p = jnp.concatenate([ew, jnp.zeros((pad,), jnp.float32)])
    src3 = src_p.reshape(_NS, _CPP, _K)
    dst3 = dst_p.reshape(_NS, _CPP, _K)
    ew3 = ew_p.reshape(_NS, _CPP, _K)
    dst2 = dst_p.reshape(_NS, 1, _EPT)
    ew2 = ew_p.reshape(_NS, 1, _EPT)

    dis_full = _deg_kernel(dst2, ew2)
    dis_col = dis_full[:_N].reshape(_N, 1)

    fshape = jax.ShapeDtypeStruct((_N, _D), jnp.float32)
    bshape = jax.ShapeDtypeStruct((_N, _D), jnp.bfloat16)

    g1, g1b = pl.pallas_call(
        _tc1_body,
        grid=(_G,),
        in_specs=[_bs_rows_f, _bs_w, _bs_dis],
        out_specs=[_bs_rows_f, _bs_rows_b],
        out_shape=[fshape, bshape],
    )(x, W1, dis_col)

    p1 = _msg_kernel(g1b, src3, dst3, ew3)

    g2, g2b = pl.pallas_call(
        _tc2_body,
        grid=(_G,),
        in_specs=[_bs_p, _bs_rows_f, _bs_dis, _bs_bias, _bs_w],
        out_specs=[_bs_rows_f, _bs_rows_b],
        out_shape=[fshape, bshape],
    )(p1, g1, dis_col, b1.reshape(1, _D), W2)

    p2 = _msg_kernel(g2b, src3, dst3, ew3)

    out = pl.pallas_call(
        _tc3_body,
        grid=(_G,),
        in_specs=[_bs_p, _bs_rows_f, _bs_dis, _bs_bias],
        out_specs=_bs_rows_f,
        out_shape=fshape,
    )(p2, g2, dis_col, b2.reshape(1, _D))

    return out


# speedup vs baseline: 1.0954x; 1.0954x over previous
import jax
import jax.numpy as jnp
from jax import lax
from jax.experimental import pallas as pl
from jax.experimental.pallas import tpu as pltpu
from jax.experimental.pallas import tpu_sc as plsc

_N = 10000
_D = 128
_E = 320000
_NC = 2
_NS = 16
_NW = _NC * _NS
_K = 128
_CPP = 160
_C0 = 104
_C1 = _CPP - _C0
_EPAD = _NS * _CPP * _K
_EPT = _EPAD // _NS
_NPAD = 10240
_RPT = _NPAD // _NS
_WB_CHUNKS = ((0, 128), (128, 128), (256, 128), (384, 128), (512, 128))


def _deg_body(dst_hbm, ew_hbm, src_hbm, dis_hbm, wp_hbm,
              dst_v, ew_v, src_v, part_v, shared, dis_sh, acc_v, tmp_v, dis_v):
    cid = lax.axis_index("c")
    sid = lax.axis_index("s")

    @pl.when(cid == 0)
    def _():
        pltpu.sync_copy(dst_hbm.at[sid, 0], dst_v)
        pltpu.sync_copy(ew_hbm.at[sid, 0], ew_v)
        pltpu.sync_copy(src_hbm.at[sid, 0], src_v)
        zeros = jnp.zeros((16,), jnp.float32)

        def zero_body(i, _):
            part_v[pl.ds(i * 16, 16)] = zeros
            return 0

        lax.fori_loop(0, _NPAD // 16, zero_body, 0)

        def scat_body(g, _):
            idx = dst_v[pl.ds(g * 16, 16)]
            w = ew_v[pl.ds(g * 16, 16)]
            plsc.addupdate_scatter(part_v, [idx], w)
            return 0

        lax.fori_loop(0, _EPT // 16, scat_body, 0)

        pltpu.sync_copy(part_v, shared.at[sid])
        plsc.subcore_barrier()

        base = sid * _RPT
        pltpu.sync_copy(shared.at[0, pl.ds(base, _RPT)], acc_v)
        for k in range(1, _NS):
            pltpu.sync_copy(shared.at[k, pl.ds(base, _RPT)], tmp_v)

            def add_body(i, _):
                sl = pl.ds(i * 16, 16)
                acc_v[sl] = acc_v[sl] + tmp_v[sl]
                return 0

            lax.fori_loop(0, _RPT // 16, add_body, 0)

        def newton_body(i, _):
            sl = pl.ds(i * 16, 16)
            xv = acc_v[sl] + 1.0
            bits = plsc.bitcast(xv, jnp.int32)
            y = plsc.bitcast(
                jnp.int32(0x5F3759DF) - lax.shift_right_logical(bits, 1),
                jnp.float32,
            )
            half = xv * 0.5
            for _u in range(3):
                y = y * (1.5 - half * y * y)
            acc_v[sl] = y
            return 0

        lax.fori_loop(0, _RPT // 16, newton_body, 0)
        pltpu.sync_copy(acc_v, dis_hbm.at[pl.ds(base, _RPT)])

        pltpu.sync_copy(acc_v, dis_sh.at[pl.ds(base, _RPT)])
        plsc.subcore_barrier()
        pltpu.sync_copy(dis_sh, dis_v)

        def norm_body(g, _):
            sl = pl.ds(g * 16, 16)
            a = plsc.load_gather(dis_v, [src_v[sl]])
            d = plsc.load_gather(dis_v, [dst_v[sl]])
            ew_v[sl] = ew_v[sl] * a * d
            return 0

        lax.fori_loop(0, _EPT // 16, norm_body, 0)
        pltpu.sync_copy(ew_v, wp_hbm.at[sid, 0])


_deg_kernel = pl.kernel(
    _deg_body,
    out_type=[jax.ShapeDtypeStruct((_NPAD,), jnp.float32),
              jax.ShapeDtypeStruct((_NS, 1, _EPT), jnp.float32)],
    mesh=plsc.VectorSubcoreMesh(core_axis_name="c", subcore_axis_name="s"),
    scratch_types=[
        pltpu.VMEM((_EPT,), jnp.int32),
        pltpu.VMEM((_EPT,), jnp.float32),
        pltpu.VMEM((_EPT,), jnp.int32),
        pltpu.VMEM((_NPAD,), jnp.float32),
        pltpu.VMEM_SHARED((_NS, _NPAD), jnp.float32),
        pltpu.VMEM_SHARED((_NPAD,), jnp.float32),
        pltpu.VMEM((_RPT,), jnp.float32),
        pltpu.VMEM((_RPT,), jnp.float32),
        pltpu.VMEM((_NPAD,), jnp.float32),
    ],
    compiler_params=pltpu.CompilerParams(needs_layout_passes=False),
)


def _bf16_splat(v):
    c16 = jnp.full((16,), 16, jnp.uint32)
    one = jnp.full((16,), 1, jnp.uint32)
    rnd = jnp.full((16,), 0x7FFF, jnp.uint32)
    vu = plsc.bitcast(v, jnp.uint32)
    vr = lax.shift_right_logical(
        vu + rnd + (lax.shift_right_logical(vu, c16) & one), c16)
    word = vr | lax.shift_left(vr, c16)
    return plsc.bitcast(word, jnp.bfloat16)


def _msg_body(g_hbm, src_hbm, dst_hbm, ew_hbm, out_hbm,
              src_v, dst_v, ew_v, buf, acc, gs0, gs1, ss0, ss1):
    cid = lax.axis_index("c")
    sid = lax.axis_index("s")
    gsem = (gs0, gs1)
    ssem = (ss0, ss1)

    zeros = jnp.zeros((32,), jnp.bfloat16)

    def zb(r, _):
        for cc in range(_D // 32):
            buf[0, r, pl.ds(cc * 32, 32)] = zeros
        return 0

    lax.fori_loop(0, _K, zb, 0)

    zbase = sid * _RPT
    for off, n in _WB_CHUNKS:
        pltpu.sync_copy(buf.at[0, pl.ds(0, n)], acc.at[pl.ds(zbase + off, n)])
    plsc.subcore_barrier()

    def _run(base, nc):
        pltpu.sync_copy(src_hbm.at[sid, pl.ds(base, nc)],
                        src_v.at[pl.ds(0, nc)])
        pltpu.sync_copy(dst_hbm.at[sid, pl.ds(base, nc)],
                        dst_v.at[pl.ds(0, nc)])
        pltpu.sync_copy(ew_hbm.at[sid, pl.ds(base, nc)],
                        ew_v.at[pl.ds(0, nc)])

        def _gather(c, b, sem):
            return pltpu.async_copy(g_hbm.at[src_v.at[c]], buf.at[b], sem)

        _gather(0, 0, gsem[0])
        _gather(1, 1, gsem[1])

        def chunk_body(i, _):
            for b in range(2):
                c = 2 * i + b
                pltpu.make_async_copy(g_hbm.at[src_v.at[c]],
                                      buf.at[b], gsem[b]).wait()
                cvec = jnp.full((16,), c, jnp.int32)

                def row_body(r, _):
                    ewf = plsc.load_gather(
                        ew_v, [cvec, jnp.full((16,), r, jnp.int32)])
                    ewb = _bf16_splat(ewf)
                    for k in range(_D // 32):
                        sl = pl.ds(32 * k, 32)
                        buf[b, r, sl] = buf[b, r, sl] * ewb
                    return 0

                lax.fori_loop(0, _K, row_body, 0)
                pltpu.async_copy(buf.at[b], acc.at[dst_v.at[c]],
                                 ssem[b], add=True)

                @pl.when(i < nc // 2 - 1)
                def _():
                    pltpu.make_async_copy(buf.at[b], acc.at[dst_v.at[c]],
                                          ssem[b]).wait()
                    _gather(c + 2, b, gsem[b])

            return 0

        lax.fori_loop(0, nc // 2, chunk_body, 0)
        for b in range(2):
            c = nc - 2 + b
            pltpu.make_async_copy(buf.at[b], acc.at[dst_v.at[c]],
                                  ssem[b]).wait()

    @pl.when(cid == 0)
    def _():
        _run(0, _C0)

    @pl.when(cid == 1)
    def _():
        _run(_C0, _C1)

    plsc.subcore_barrier()

    for off, n in _WB_CHUNKS:
        pltpu.sync_copy(acc.at[pl.ds(zbase + off, n)],
                        out_hbm.at[cid, pl.ds(zbase + off, n)])


_msg_kernel = pl.kernel(
    _msg_body,
    out_type=jax.ShapeDtypeStruct((_NC, _NPAD, _D), jnp.bfloat16),
    mesh=plsc.VectorSubcoreMesh(core_axis_name="c", subcore_axis_name="s"),
    scratch_types=[
        pltpu.VMEM((_C0, _K), jnp.int32),
        pltpu.VMEM((_C0, _K), jnp.int32),
        pltpu.VMEM((_C0, _K), jnp.float32),
        pltpu.VMEM((2, _K, _D), jnp.bfloat16),
        pltpu.VMEM_SHARED((_NPAD, _D), jnp.bfloat16),
        pltpu.SemaphoreType.DMA,
        pltpu.SemaphoreType.DMA,
        pltpu.SemaphoreType.DMA,
        pltpu.SemaphoreType.DMA,
    ],
    compiler_params=pltpu.CompilerParams(needs_layout_passes=False,
                                         use_tc_tiling_on_sc=False),
)


_GB = 2000
_G = _N // _GB


def _tc1_body(x_ref, w_ref, h_ref, hb_ref):
    h = jnp.dot(x_ref[...], w_ref[...], preferred_element_type=jnp.float32)
    h_ref[...] = h
    hb_ref[...] = h.astype(jnp.bfloat16)


def _tc2_body(p_ref, h_ref, dis_ref, b_ref, w_ref, h2_ref, h2b_ref):
    p = (p_ref[0].astype(jnp.float32) + p_ref[1].astype(jnp.float32))
    dis = dis_ref[...]
    s = p + h_ref[...] * (dis * dis) + b_ref[...]
    t = jnp.maximum(s, 0.0)
    h2 = jnp.dot(t, w_ref[...], preferred_element_type=jnp.float32)
    h2_ref[...] = h2
    h2b_ref[...] = h2.astype(jnp.bfloat16)


def _tc3_body(p_ref, h_ref, dis_ref, b_ref, out_ref):
    p = (p_ref[0].astype(jnp.float32) + p_ref[1].astype(jnp.float32))
    dis = dis_ref[...]
    out_ref[...] = p + h_ref[...] * (dis * dis) + b_ref[...]


_bs_rows_f = pl.BlockSpec((_GB, _D), lambda i: (i, 0))
_bs_rows_b = pl.BlockSpec((_GB, _D), lambda i: (i, 0))
_bs_dis = pl.BlockSpec((_GB, 1), lambda i: (i, 0))
_bs_w = pl.BlockSpec((_D, _D), lambda i: (0, 0))
_bs_bias = pl.BlockSpec((1, _D), lambda i: (0, 0))
_bs_p = pl.BlockSpec((_NC, _GB, _D), lambda i: (0, i, 0))


def kernel(x, edge_index, edge_weight, W1, b1, W2, b2):
    src = edge_index[0].astype(jnp.int32)
    dst = edge_index[1].astype(jnp.int32)
    ew = edge_weight.astype(jnp.float32)

    pad = _EPAD - _E
    src_p = jnp.concatenate([src, jnp.zeros((pad,), jnp.int32)])
    dst_p = jnp.concatenate([dst, jnp.zeros((pad,), jnp.int32)])
    ew_p = jnp.concatenate([ew, jnp.zeros((pad,), jnp.float32)])
    src3 = src_p.reshape(_NS, _CPP, _K)
    dst3 = dst_p.reshape(_NS, _CPP, _K)
    src2 = src_p.reshape(_NS, 1, _EPT)
    dst2 = dst_p.reshape(_NS, 1, _EPT)
    ew2 = ew_p.reshape(_NS, 1, _EPT)

    dis_full, wp = _deg_kernel(dst2, ew2, src2)
    dis_col = dis_full[:_N].reshape(_N, 1)
    wp3 = wp.reshape(_NS, _CPP, _K)

    fshape = jax.ShapeDtypeStruct((_N, _D), jnp.float32)
    bshape = jax.ShapeDtypeStruct((_N, _D), jnp.bfloat16)

    g1, g1b = pl.pallas_call(
        _tc1_body,
        grid=(_G,),
        in_specs=[_bs_rows_f, _bs_w],
        out_specs=[_bs_rows_f, _bs_rows_b],
        out_shape=[fshape, bshape],
    )(x, W1)

    p1 = _msg_kernel(g1b, src3, dst3, wp3)

    g2, g2b = pl.pallas_call(
        _tc2_body,
        grid=(_G,),
        in_specs=[_bs_p, _bs_rows_f, _bs_dis, _bs_bias, _bs_w],
        out_specs=[_bs_rows_f, _bs_rows_b],
        out_shape=[fshape, bshape],
    )(p1, g1, dis_col, b1.reshape(1, _D), W2)

    p2 = _msg_kernel(g2b, src3, dst3, wp3)

    out = pl.pallas_call(
        _tc3_body,
        grid=(_G,),
        in_specs=[_bs_p, _bs_rows_f, _bs_dis, _bs_bias],
        out_specs=_bs_rows_f,
        out_shape=fshape,
    )(p2, g2, dis_col, b2.reshape(1, _D))

    return out
